# Initial kernel scaffold; baseline (speedup 1.0000x reference)
import jax
import jax.numpy as jnp
from jax.experimental import pallas as pl


def kernel(qid_table, uid_table, click_table, vid_table, pos_table, W_q, a_src_q, a_dst_q, b_q, W_u, a_src_u, a_dst_u, b_u, qid_edge_index, uid_edge_index, QIDS, UIDS, VIDS, CLICKS):
    raise NotImplementedError("write your pallas kernel here")


# speedup vs baseline: 28.1972x; 28.1972x over previous
import functools

import jax
import jax.numpy as jnp
from jax import lax
from jax.experimental import pallas as pl
from jax.experimental.pallas import tpu as pltpu
from jax.experimental.pallas import tpu_sc as plsc

SLOPE = 0.2
CHUNK = 80
NS = 16

_MESH = plsc.VectorSubcoreMesh(core_axis_name="c", subcore_axis_name="s")
_PARAMS = pltpu.CompilerParams(needs_layout_passes=False)


def _blockdiag(a):
    H, C = a.shape
    eye = jnp.eye(H, dtype=a.dtype)
    return (a[:, :, None] * eye[:, None, :]).reshape(H * C, H)


def _dense_tc(xs, Ws, ABs):
    G, N, D = xs.shape
    RB = 2000

    def body(x_ref, w_ref, ab_ref, h_ref, abo_ref):
        x = x_ref[0]
        h = jnp.dot(x, w_ref[0], preferred_element_type=jnp.float32)
        h_ref[0] = h
        abo_ref[0] = jnp.dot(h, ab_ref[0], preferred_element_type=jnp.float32)

    return pl.pallas_call(
        body,
        grid=(G, N // RB),
        in_specs=[
            pl.BlockSpec((1, RB, D), lambda g, i: (g, i, 0)),
            pl.BlockSpec((1, D, D), lambda g, i: (g, 0, 0)),
            pl.BlockSpec((1, D, 8), lambda g, i: (g, 0, 0)),
        ],
        out_specs=[
            pl.BlockSpec((1, RB, D), lambda g, i: (g, i, 0)),
            pl.BlockSpec((1, RB, 8), lambda g, i: (g, i, 0)),
        ],
        out_shape=[
            jax.ShapeDtypeStruct((G, N, D), jnp.float32),
            jax.ShapeDtypeStruct((G, N, 8), jnp.float32),
        ],
    )(xs, Ws, ABs)


def _edge_w_sc(ab_q, src_q, dst_q, ab_u, src_u, dst_u):
    E = src_q.shape[0]
    H = 4
    NA = ab_q.shape[0]
    ept = E // NS
    n_echunks = ept // CHUNK

    @functools.partial(
        pl.kernel,
        out_type=(
            jax.ShapeDtypeStruct((E, H), jnp.float32),
            jax.ShapeDtypeStruct((E, H), jnp.float32),
        ),
        mesh=_MESH,
        compiler_params=_PARAMS,
        scratch_types=[
            pltpu.VMEM((NA,), jnp.float32),
            pltpu.VMEM((CHUNK,), jnp.int32),
            pltpu.VMEM((CHUNK,), jnp.int32),
            pltpu.VMEM((CHUNK, H), jnp.float32),
        ],
    )
    def k(abq_hbm, sq_hbm, dq_hbm, abu_hbm, su_hbm, du_hbm,
          wq_hbm, wu_hbm, abv, srcv, dstv, wout):
        cid = lax.axis_index("c")
        sid = lax.axis_index("s")
        lanes = lax.iota(jnp.int32, 16)

        def run(ab_hbm, src_hbm, dst_hbm, w_hbm):
            pltpu.sync_copy(ab_hbm, abv)
            ebase = sid * ept

            def echunk(ci, _):
                off = ebase + ci * CHUNK
                pltpu.sync_copy(src_hbm.at[pl.ds(off, CHUNK)], srcv)
                pltpu.sync_copy(dst_hbm.at[pl.ds(off, CHUNK)], dstv)
                for g in range(CHUNK // 16):
                    e16 = lanes + g * 16
                    s16 = srcv[pl.ds(g * 16, 16)]
                    d16 = dstv[pl.ds(g * 16, 16)]
                    for hh in range(H):
                        a_s = plsc.load_gather(abv, [s16 * 8 + hh])
                        a_d = plsc.load_gather(abv, [d16 * 8 + (H + hh)])
                        ee = a_s + a_d
                        ee = jnp.where(ee > 0, ee, SLOPE * ee)
                        w = jnp.exp(ee)
                        plsc.store_scatter(
                            wout, [e16, jnp.full((16,), hh, jnp.int32)], w)
                pltpu.sync_copy(wout, w_hbm.at[pl.ds(off, CHUNK)])
                return 0
            lax.fori_loop(0, n_echunks, echunk, 0)

        @pl.when(cid == 0)
        def _():
            run(abq_hbm, sq_hbm, dq_hbm, wq_hbm)

        @pl.when(cid == 1)
        def _():
            run(abu_hbm, su_hbm, du_hbm, wu_hbm)

    return k(ab_q, src_q, dst_q, ab_u, src_u, dst_u)


def _aggregate_sc(h_q, w_q, b_q, src_q, dst_q, h_u, w_u, b_u, src_u, dst_u):
    N, D = h_q.shape
    E = src_q.shape[0]
    H = 4
    C = D // H
    ept = E // NS
    n_echunks = ept // CHUNK
    row_chunks = N // CHUNK
    iters = -(-row_chunks // NS)
    DR = (N * H + 127) // 128
    DRP = -(-DR // CHUNK) * CHUNK

    @functools.partial(
        pl.kernel,
        out_type=(
            jax.ShapeDtypeStruct((N, D), jnp.float32),
            jax.ShapeDtypeStruct((N, D), jnp.float32),
        ),
        mesh=_MESH,
        compiler_params=_PARAMS,
        scratch_types=[
            pltpu.VMEM_SHARED((N, D), jnp.float32),
            pltpu.VMEM_SHARED((DRP, 128), jnp.float32),
            pltpu.VMEM((CHUNK,), jnp.int32),
            pltpu.VMEM((CHUNK,), jnp.int32),
            pltpu.VMEM((CHUNK,), jnp.int32),
            pltpu.VMEM((CHUNK, D), jnp.float32),
            pltpu.VMEM((CHUNK, D), jnp.float32),
            pltpu.VMEM((CHUNK, 128), jnp.float32),
            pltpu.VMEM((CHUNK, H), jnp.float32),
            pltpu.VMEM((16, 128), jnp.float32),
            pltpu.VMEM((D,), jnp.float32),
            pltpu.SemaphoreType.DMA,
        ],
    )
    def k(hq_hbm, wq_hbm, bq_hbm, sq_hbm, dq_hbm,
          hu_hbm, wu_hbm, bu_hbm, su_hbm, du_hbm,
          outq_hbm, outu_hbm,
          acc, accd, srcv, dstv, drow, hbuf, msg, mden, wch,
          dbuf, biasv, sem):
        cid = lax.axis_index("c")
        sid = lax.axis_index("s")
        zero16 = jnp.zeros((16,), jnp.float32)
        lanes = lax.iota(jnp.int32, 16)

        def run(h_hbm, w_hbm, b_hbm, src_hbm, dst_hbm, out_hbm):
            pltpu.sync_copy(b_hbm, biasv)

            def zmsg(e, _):
                for kk in range(D // 16):
                    msg[e, pl.ds(kk * 16, 16)] = zero16
                for kk in range(128 // 16):
                    mden[e, pl.ds(kk * 16, 16)] = zero16
                return 0
            lax.fori_loop(0, CHUNK, zmsg, 0)

            def zacc(i, _):
                c = sid + i * NS

                @pl.when(c < row_chunks)
                def _():
                    pltpu.sync_copy(msg, acc.at[pl.ds(c * CHUNK, CHUNK)])
                return 0
            lax.fori_loop(0, iters, zacc, 0)

            @pl.when(sid < DRP // CHUNK)
            def _():
                pltpu.sync_copy(msg, accd.at[pl.ds(sid * CHUNK, CHUNK)])
            plsc.subcore_barrier()

            ebase = sid * ept

            def echunk(ci, _):
                off = ebase + ci * CHUNK
                pltpu.sync_copy(src_hbm.at[pl.ds(off, CHUNK)], srcv)
                pltpu.sync_copy(dst_hbm.at[pl.ds(off, CHUNK)], dstv)
                pltpu.sync_copy(w_hbm.at[pl.ds(off, CHUNK)], wch)
                pltpu.async_copy(h_hbm.at[srcv], hbuf, sem).wait()
                for g in range(CHUNK // 16):
                    e16 = lanes + g * 16
                    d16 = dstv[pl.ds(g * 16, 16)]
                    drow[pl.ds(g * 16, 16)] = d16 >> 5
                    colb = (d16 & 31) * H
                    for hh in range(H):
                        w = plsc.load_gather(
                            wch, [e16, jnp.full((16,), hh, jnp.int32)])
                        plsc.store_scatter(mden, [e16, colb + hh], w)

                def medge(e, _):
                    ev = jnp.full((16,), e, jnp.int32)
                    for hh in range(H):
                        wv = plsc.load_gather(
                            wch, [ev, jnp.full((16,), hh, jnp.int32)])
                        for half in range(C // 16):
                            col = hh * C + half * 16
                            msg[e, pl.ds(col, 16)] = hbuf[e, pl.ds(col, 16)] * wv
                    return 0
                lax.fori_loop(0, CHUNK, medge, 0)
                pltpu.sync_copy(msg, acc.at[dstv], add=True)
                pltpu.sync_copy(mden, accd.at[drow], add=True)
                for g in range(CHUNK // 16):
                    e16 = lanes + g * 16
                    d16 = dstv[pl.ds(g * 16, 16)]
                    colb = (d16 & 31) * H
                    for hh in range(H):
                        plsc.store_scatter(mden, [e16, colb + hh], zero16)
                return 0
            lax.fori_loop(0, n_echunks, echunk, 0)
            plsc.subcore_barrier()

            def nchunk(i, _):
                c = sid + i * NS

                @pl.when(c < row_chunks)
                def _():
                    base = c * CHUNK
                    pltpu.sync_copy(acc.at[pl.ds(base, CHUNK)], msg)
                    r0a = ((base // 32) // 8) * 8
                    pltpu.sync_copy(accd.at[pl.ds(r0a, 16)], dbuf)

                    def node(n, _):
                        p = (base + n) * H - r0a * 128
                        for hh in range(H):
                            rv = jnp.full((16,), (p + hh) >> 7, jnp.int32)
                            cv = jnp.full((16,), (p + hh) & 127, jnp.int32)
                            dv = plsc.load_gather(dbuf, [rv, cv])
                            invv = 1.0 / (dv + 1e-16)
                            for half in range(C // 16):
                                col = hh * C + half * 16
                                v = msg[n, pl.ds(col, 16)] * invv \
                                    + biasv[pl.ds(col, 16)]
                                hbuf[n, pl.ds(col, 16)] = jnp.maximum(v, 0.0)
                        return 0
                    lax.fori_loop(0, CHUNK, node, 0)
                    pltpu.sync_copy(hbuf, out_hbm.at[pl.ds(base, CHUNK)])
                return 0
            lax.fori_loop(0, iters, nchunk, 0)

        @pl.when(cid == 0)
        def _():
            run(hq_hbm, wq_hbm, bq_hbm, sq_hbm, dq_hbm, outq_hbm)

        @pl.when(cid == 1)
        def _():
            run(hu_hbm, wu_hbm, bu_hbm, su_hbm, du_hbm, outu_hbm)

    return k(h_q, w_q, b_q, src_q, dst_q, h_u, w_u, b_u, src_u, dst_u)


def _session_gather_sc(pq, pu, vid_table, click_table, pos_table,
                       qidx, uidx, vidx, cidx, pidx):
    N, D = pq.shape
    BS = qidx.shape[0]
    per_tile = BS // (2 * NS)
    n_chunks = per_tile // CHUNK

    @functools.partial(
        pl.kernel,
        out_type=(
            jax.ShapeDtypeStruct((BS, D), jnp.float32),
            jax.ShapeDtypeStruct((BS, D), jnp.float32),
            jax.ShapeDtypeStruct((BS, 16), jnp.float32),
            jax.ShapeDtypeStruct((BS, 16), jnp.float32),
            jax.ShapeDtypeStruct((BS, 16), jnp.float32),
        ),
        mesh=_MESH,
        compiler_params=_PARAMS,
        scratch_types=[
            pltpu.VMEM((CHUNK,), jnp.int32),
            pltpu.VMEM((CHUNK, D), jnp.float32),
            pltpu.VMEM((CHUNK, 16), jnp.float32),
            pltpu.VMEM((16, 16), jnp.float32),
            pltpu.SemaphoreType.DMA,
        ],
    )
    def k(pq_hbm, pu_hbm, vid_hbm, click_hbm, pos_hbm,
          qi_hbm, ui_hbm, vi_hbm, ci_hbm, pi_hbm,
          oq, ou, ov, oc, op_,
          idxv, bigbuf, smallbuf, tabv, sem):
        wid = lax.axis_index("s") * 2 + lax.axis_index("c")
        base = wid * per_tile
        lanes = lax.iota(jnp.int32, 16)

        def gather_big(tab, idx_hbm, out_hbm):
            def step(j, _):
                off = base + j * CHUNK
                pltpu.sync_copy(idx_hbm.at[pl.ds(off, CHUNK)], idxv)
                pltpu.async_copy(tab.at[idxv], bigbuf, sem).wait()
                pltpu.sync_copy(bigbuf, out_hbm.at[pl.ds(off, CHUNK)])
                return 0
            lax.fori_loop(0, n_chunks, step, 0)

        def gather_small(tab, rows, idx_hbm, out_hbm):
            pltpu.sync_copy(tab, tabv.at[pl.ds(0, rows)])

            def step(j, _):
                off = base + j * CHUNK
                pltpu.sync_copy(idx_hbm.at[pl.ds(off, CHUNK)], idxv)

                def row(r, _):
                    ispl = plsc.load_gather(
                        idxv, [jnp.full((16,), r, jnp.int32)])
                    smallbuf[r, pl.ds(0, 16)] = plsc.load_gather(
                        tabv, [ispl, lanes])
                    return 0
                lax.fori_loop(0, CHUNK, row, 0)
                pltpu.sync_copy(smallbuf, out_hbm.at[pl.ds(off, CHUNK)])
                return 0
            lax.fori_loop(0, n_chunks, step, 0)

        gather_big(pq_hbm, qi_hbm, oq)
        gather_big(pu_hbm, ui_hbm, ou)
        gather_small(vid_hbm, vid_hbm.shape[0], vi_hbm, ov)
        gather_small(click_hbm, click_hbm.shape[0], ci_hbm, oc)
        gather_small(pos_hbm, pos_hbm.shape[0], pi_hbm, op_)

    return k(pq, pu, vid_table, click_table, pos_table,
             qidx, uidx, vidx, cidx, pidx)


def kernel(qid_table, uid_table, click_table, vid_table, pos_table,
           W_q, a_src_q, a_dst_q, b_q, W_u, a_src_u, a_dst_u, b_u,
           qid_edge_index, uid_edge_index, QIDS, UIDS, VIDS, CLICKS):
    B, S = QIDS.shape
    D = qid_table.shape[1]
    xs = jnp.stack([qid_table, uid_table])
    Ws = jnp.stack([W_q, W_u])
    ABs = jnp.stack([
        jnp.concatenate([_blockdiag(a_src_q), _blockdiag(a_dst_q)], axis=1),
        jnp.concatenate([_blockdiag(a_src_u), _blockdiag(a_dst_u)], axis=1),
    ])
    hs, abs_ = _dense_tc(xs, Ws, ABs)

    w_q, w_u = _edge_w_sc(
        abs_[0].reshape(-1), qid_edge_index[0], qid_edge_index[1],
        abs_[1].reshape(-1), uid_edge_index[0], uid_edge_index[1])

    pq, pu = _aggregate_sc(
        hs[0], w_q, b_q, qid_edge_index[0], qid_edge_index[1],
        hs[1], w_u, b_u, uid_edge_index[0], uid_edge_index[1])

    pidx = jnp.tile(jnp.arange(10, dtype=jnp.int32), (B * S) // 10)
    qe, ue, ve, ce, pe = _session_gather_sc(
        pq, pu, vid_table, click_table, pos_table,
        QIDS.reshape(-1), UIDS.reshape(-1), VIDS.reshape(-1),
        CLICKS.reshape(-1), pidx)
    return (qe.reshape(B, S, D), ue.reshape(B, S, D),
            ve.reshape(B, S, 16), ce.reshape(B, S, 16),
            pe.reshape(B, S, 16))


# speedup vs baseline: 40.0410x; 1.4200x over previous
import functools

import jax
import jax.numpy as jnp
from jax import lax
from jax.experimental import pallas as pl
from jax.experimental.pallas import tpu as pltpu
from jax.experimental.pallas import tpu_sc as plsc

SLOPE = 0.2
CHUNK = 80
NS = 16

_MESH = plsc.VectorSubcoreMesh(core_axis_name="c", subcore_axis_name="s")
_PARAMS = pltpu.CompilerParams(needs_layout_passes=False)


def _blockdiag(a):
    H, C = a.shape
    eye = jnp.eye(H, dtype=a.dtype)
    return (a[:, :, None] * eye[:, None, :]).reshape(H * C, H)


def _dense_tc(xs, Ws, ABs):
    G, N, D = xs.shape
    RB = 2000

    def body(x_ref, w_ref, ab_ref, h_ref, abo_ref):
        x = x_ref[0]
        h = jnp.dot(x, w_ref[0], preferred_element_type=jnp.float32)
        h_ref[0] = h
        abo_ref[0] = jnp.dot(h, ab_ref[0], preferred_element_type=jnp.float32)

    return pl.pallas_call(
        body,
        grid=(G, N // RB),
        in_specs=[
            pl.BlockSpec((1, RB, D), lambda g, i: (g, i, 0)),
            pl.BlockSpec((1, D, D), lambda g, i: (g, 0, 0)),
            pl.BlockSpec((1, D, 8), lambda g, i: (g, 0, 0)),
        ],
        out_specs=[
            pl.BlockSpec((1, RB, D), lambda g, i: (g, i, 0)),
            pl.BlockSpec((1, RB, 8), lambda g, i: (g, i, 0)),
        ],
        out_shape=[
            jax.ShapeDtypeStruct((G, N, D), jnp.float32),
            jax.ShapeDtypeStruct((G, N, 8), jnp.float32),
        ],
    )(xs, Ws, ABs)


def _edge_w_sc(ab_q, src_q, dst_q, ab_u, src_u, dst_u):
    E = src_q.shape[0]
    H = 4
    NA = ab_q.shape[0]
    ept = E // NS
    WB = 2000
    n_batches = ept // WB

    @functools.partial(
        pl.kernel,
        out_type=(
            jax.ShapeDtypeStruct((E * H,), jnp.float32),
            jax.ShapeDtypeStruct((E * H,), jnp.float32),
        ),
        mesh=_MESH,
        compiler_params=_PARAMS,
        scratch_types=[
            pltpu.VMEM((NA,), jnp.float32),
            pltpu.VMEM((WB,), jnp.int32),
            pltpu.VMEM((WB,), jnp.int32),
            pltpu.VMEM((WB * H,), jnp.float32),
        ],
    )
    def k(abq_hbm, sq_hbm, dq_hbm, abu_hbm, su_hbm, du_hbm,
          wq_hbm, wu_hbm, abv, srcv, dstv, wout):
        cid = lax.axis_index("c")
        sid = lax.axis_index("s")
        lanes = lax.iota(jnp.int32, 16)

        def run(ab_hbm, src_hbm, dst_hbm, w_hbm):
            pltpu.sync_copy(ab_hbm, abv)
            ebase = sid * ept

            def ebatch(bi, _):
                off = ebase + bi * WB
                pltpu.sync_copy(src_hbm.at[pl.ds(off, WB)], srcv)
                pltpu.sync_copy(dst_hbm.at[pl.ds(off, WB)], dstv)

                def group(g, _):
                    e16 = lanes + g * 16
                    s16 = srcv[pl.ds(g * 16, 16)]
                    d16 = dstv[pl.ds(g * 16, 16)]
                    for hh in range(H):
                        a_s = plsc.load_gather(abv, [s16 * 8 + hh])
                        a_d = plsc.load_gather(abv, [d16 * 8 + (H + hh)])
                        ee = a_s + a_d
                        ee = jnp.where(ee > 0, ee, SLOPE * ee)
                        w = jnp.exp(ee)
                        plsc.store_scatter(wout, [e16 * H + hh], w)
                    return 0
                lax.fori_loop(0, WB // 16, group, 0)
                pltpu.sync_copy(wout, w_hbm.at[pl.ds(off * H, WB * H)])
                return 0
            lax.fori_loop(0, n_batches, ebatch, 0)

        @pl.when(cid == 0)
        def _():
            run(abq_hbm, sq_hbm, dq_hbm, wq_hbm)

        @pl.when(cid == 1)
        def _():
            run(abu_hbm, su_hbm, du_hbm, wu_hbm)

    return k(ab_q, src_q, dst_q, ab_u, src_u, dst_u)


def _aggregate_sc(h_q, w_q, b_q, src_q, dst_q, h_u, w_u, b_u, src_u, dst_u):
    N, D = h_q.shape
    E = src_q.shape[0]
    H = 4
    C = D // H
    ept = E // NS
    EB = 2000
    n_batches = ept // EB
    n_sub = EB // CHUNK
    row_chunks = N // CHUNK
    iters = -(-row_chunks // NS)
    DR = (N * H + 127) // 128
    DRP = -(-DR // CHUNK) * CHUNK

    @functools.partial(
        pl.kernel,
        out_type=(
            jax.ShapeDtypeStruct((N, D), jnp.float32),
            jax.ShapeDtypeStruct((N, D), jnp.float32),
        ),
        mesh=_MESH,
        compiler_params=_PARAMS,
        scratch_types=[
            pltpu.VMEM_SHARED((N, D), jnp.float32),
            pltpu.VMEM_SHARED((DRP, 128), jnp.float32),
            pltpu.VMEM((EB,), jnp.int32),
            pltpu.VMEM((EB,), jnp.int32),
            pltpu.VMEM((EB * H,), jnp.float32),
            pltpu.VMEM((CHUNK,), jnp.int32),
            pltpu.VMEM((CHUNK,), jnp.int32),
            pltpu.VMEM((CHUNK, D), jnp.float32),
            pltpu.VMEM((CHUNK, D), jnp.float32),
            pltpu.VMEM((CHUNK, 128), jnp.float32),
            pltpu.VMEM((16, 128), jnp.float32),
            pltpu.VMEM((D,), jnp.float32),
            pltpu.SemaphoreType.DMA,
        ],
    )
    def k(hq_hbm, wq_hbm, bq_hbm, sq_hbm, dq_hbm,
          hu_hbm, wu_hbm, bu_hbm, su_hbm, du_hbm,
          outq_hbm, outu_hbm,
          acc, accd, srcb, dstb, wb, dstv, drow, hbuf, msg, mden,
          dbuf, biasv, sem):
        cid = lax.axis_index("c")
        sid = lax.axis_index("s")
        zero16 = jnp.zeros((16,), jnp.float32)
        lanes = lax.iota(jnp.int32, 16)

        def run(h_hbm, w_hbm, b_hbm, src_hbm, dst_hbm, out_hbm):
            pltpu.sync_copy(b_hbm, biasv)

            def zmsg(e, _):
                for kk in range(D // 16):
                    msg[e, pl.ds(kk * 16, 16)] = zero16
                for kk in range(128 // 16):
                    mden[e, pl.ds(kk * 16, 16)] = zero16
                return 0
            lax.fori_loop(0, CHUNK, zmsg, 0)

            def zacc(i, _):
                c = sid + i * NS

                @pl.when(c < row_chunks)
                def _():
                    pltpu.sync_copy(msg, acc.at[pl.ds(c * CHUNK, CHUNK)])
                return 0
            lax.fori_loop(0, iters, zacc, 0)

            @pl.when(sid < DRP // CHUNK)
            def _():
                pltpu.sync_copy(msg, accd.at[pl.ds(sid * CHUNK, CHUNK)])
            plsc.subcore_barrier()

            ebase = sid * ept

            def ebatch(bi, _):
                boff = ebase + bi * EB
                pltpu.sync_copy(src_hbm.at[pl.ds(boff, EB)], srcb)
                pltpu.sync_copy(dst_hbm.at[pl.ds(boff, EB)], dstb)
                pltpu.sync_copy(w_hbm.at[pl.ds(boff * H, EB * H)], wb)

                def sub(si, _):
                    s0 = si * CHUNK
                    pltpu.async_copy(
                        h_hbm.at[srcb.at[pl.ds(s0, CHUNK)]], hbuf, sem).wait()
                    for g in range(CHUNK // 16):
                        e16 = lanes + g * 16
                        d16 = dstb[pl.ds(s0 + g * 16, 16)]
                        dstv[pl.ds(g * 16, 16)] = d16
                        drow[pl.ds(g * 16, 16)] = d16 >> 5
                        colb = (d16 & 31) * H
                        for hh in range(H):
                            w = plsc.load_gather(wb, [(s0 + e16) * H + hh])
                            plsc.store_scatter(mden, [e16, colb + hh], w)

                    def medge(e, _):
                        for hh in range(H):
                            wv = plsc.load_gather(
                                wb, [jnp.full((16,), (s0 + e) * H + hh,
                                              jnp.int32)])
                            for half in range(C // 16):
                                col = hh * C + half * 16
                                msg[e, pl.ds(col, 16)] = \
                                    hbuf[e, pl.ds(col, 16)] * wv
                        return 0
                    lax.fori_loop(0, CHUNK, medge, 0)
                    pltpu.sync_copy(msg, acc.at[dstv], add=True)
                    pltpu.sync_copy(mden, accd.at[drow], add=True)
                    for g in range(CHUNK // 16):
                        e16 = lanes + g * 16
                        d16 = dstb[pl.ds(s0 + g * 16, 16)]
                        colb = (d16 & 31) * H
                        for hh in range(H):
                            plsc.store_scatter(mden, [e16, colb + hh], zero16)
                    return 0
                lax.fori_loop(0, n_sub, sub, 0)
                return 0
            lax.fori_loop(0, n_batches, ebatch, 0)
            plsc.subcore_barrier()

            def nchunk(i, _):
                c = sid + i * NS

                @pl.when(c < row_chunks)
                def _():
                    base = c * CHUNK
                    pltpu.sync_copy(acc.at[pl.ds(base, CHUNK)], msg)
                    r0a = ((base // 32) // 8) * 8
                    pltpu.sync_copy(accd.at[pl.ds(r0a, 16)], dbuf)

                    def node(n, _):
                        p = (base + n) * H - r0a * 128
                        for hh in range(H):
                            rv = jnp.full((16,), (p + hh) >> 7, jnp.int32)
                            cv = jnp.full((16,), (p + hh) & 127, jnp.int32)
                            dv = plsc.load_gather(dbuf, [rv, cv])
                            invv = 1.0 / (dv + 1e-16)
                            for half in range(C // 16):
                                col = hh * C + half * 16
                                v = msg[n, pl.ds(col, 16)] * invv \
                                    + biasv[pl.ds(col, 16)]
                                hbuf[n, pl.ds(col, 16)] = jnp.maximum(v, 0.0)
                        return 0
                    lax.fori_loop(0, CHUNK, node, 0)
                    pltpu.sync_copy(hbuf, out_hbm.at[pl.ds(base, CHUNK)])
                return 0
            lax.fori_loop(0, iters, nchunk, 0)

        @pl.when(cid == 0)
        def _():
            run(hq_hbm, wq_hbm, bq_hbm, sq_hbm, dq_hbm, outq_hbm)

        @pl.when(cid == 1)
        def _():
            run(hu_hbm, wu_hbm, bu_hbm, su_hbm, du_hbm, outu_hbm)

    return k(h_q, w_q, b_q, src_q, dst_q, h_u, w_u, b_u, src_u, dst_u)


def _session_gather_sc(pq, pu, vid_table, click_table, pos_table,
                       qidx, uidx, vidx, cidx, pidx):
    N, D = pq.shape
    BS = qidx.shape[0]
    per_tile = BS // (2 * NS)
    n_chunks = per_tile // CHUNK

    @functools.partial(
        pl.kernel,
        out_type=(
            jax.ShapeDtypeStruct((BS, D), jnp.float32),
            jax.ShapeDtypeStruct((BS, D), jnp.float32),
            jax.ShapeDtypeStruct((BS, 16), jnp.float32),
            jax.ShapeDtypeStruct((BS, 16), jnp.float32),
            jax.ShapeDtypeStruct((BS, 16), jnp.float32),
        ),
        mesh=_MESH,
        compiler_params=_PARAMS,
        scratch_types=[
            pltpu.VMEM((CHUNK,), jnp.int32),
            pltpu.VMEM((CHUNK, D), jnp.float32),
            pltpu.VMEM((CHUNK, 16), jnp.float32),
            pltpu.VMEM((16, 16), jnp.float32),
            pltpu.SemaphoreType.DMA,
        ],
    )
    def k(pq_hbm, pu_hbm, vid_hbm, click_hbm, pos_hbm,
          qi_hbm, ui_hbm, vi_hbm, ci_hbm, pi_hbm,
          oq, ou, ov, oc, op_,
          idxv, bigbuf, smallbuf, tabv, sem):
        wid = lax.axis_index("s") * 2 + lax.axis_index("c")
        base = wid * per_tile
        lanes = lax.iota(jnp.int32, 16)

        def gather_big(tab, idx_hbm, out_hbm):
            def step(j, _):
                off = base + j * CHUNK
                pltpu.sync_copy(idx_hbm.at[pl.ds(off, CHUNK)], idxv)
                pltpu.async_copy(tab.at[idxv], bigbuf, sem).wait()
                pltpu.sync_copy(bigbuf, out_hbm.at[pl.ds(off, CHUNK)])
                return 0
            lax.fori_loop(0, n_chunks, step, 0)

        def gather_small(tab, rows, idx_hbm, out_hbm):
            pltpu.sync_copy(tab, tabv.at[pl.ds(0, rows)])

            def step(j, _):
                off = base + j * CHUNK
                pltpu.sync_copy(idx_hbm.at[pl.ds(off, CHUNK)], idxv)

                def row(r, _):
                    ispl = plsc.load_gather(
                        idxv, [jnp.full((16,), r, jnp.int32)])
                    smallbuf[r, pl.ds(0, 16)] = plsc.load_gather(
                        tabv, [ispl, lanes])
                    return 0
                lax.fori_loop(0, CHUNK, row, 0)
                pltpu.sync_copy(smallbuf, out_hbm.at[pl.ds(off, CHUNK)])
                return 0
            lax.fori_loop(0, n_chunks, step, 0)

        gather_big(pq_hbm, qi_hbm, oq)
        gather_big(pu_hbm, ui_hbm, ou)
        gather_small(vid_hbm, vid_hbm.shape[0], vi_hbm, ov)
        gather_small(click_hbm, click_hbm.shape[0], ci_hbm, oc)
        gather_small(pos_hbm, pos_hbm.shape[0], pi_hbm, op_)

    return k(pq, pu, vid_table, click_table, pos_table,
             qidx, uidx, vidx, cidx, pidx)


def kernel(qid_table, uid_table, click_table, vid_table, pos_table,
           W_q, a_src_q, a_dst_q, b_q, W_u, a_src_u, a_dst_u, b_u,
           qid_edge_index, uid_edge_index, QIDS, UIDS, VIDS, CLICKS):
    B, S = QIDS.shape
    D = qid_table.shape[1]
    xs = jnp.stack([qid_table, uid_table])
    Ws = jnp.stack([W_q, W_u])
    ABs = jnp.stack([
        jnp.concatenate([_blockdiag(a_src_q), _blockdiag(a_dst_q)], axis=1),
        jnp.concatenate([_blockdiag(a_src_u), _blockdiag(a_dst_u)], axis=1),
    ])
    hs, abs_ = _dense_tc(xs, Ws, ABs)

    w_q, w_u = _edge_w_sc(
        abs_[0].reshape(-1), qid_edge_index[0], qid_edge_index[1],
        abs_[1].reshape(-1), uid_edge_index[0], uid_edge_index[1])

    pq, pu = _aggregate_sc(
        hs[0], w_q, b_q, qid_edge_index[0], qid_edge_index[1],
        hs[1], w_u, b_u, uid_edge_index[0], uid_edge_index[1])

    pidx = jnp.tile(jnp.arange(10, dtype=jnp.int32), (B * S) // 10)
    qe, ue, ve, ce, pe = _session_gather_sc(
        pq, pu, vid_table, click_table, pos_table,
        QIDS.reshape(-1), UIDS.reshape(-1), VIDS.reshape(-1),
        CLICKS.reshape(-1), pidx)
    return (qe.reshape(B, S, D), ue.reshape(B, S, D),
            ve.reshape(B, S, 16), ce.reshape(B, S, 16),
            pe.reshape(B, S, 16))


# speedup vs baseline: 46.1672x; 1.1530x over previous
import functools

import jax
import jax.numpy as jnp
from jax import lax
from jax.experimental import pallas as pl
from jax.experimental.pallas import tpu as pltpu
from jax.experimental.pallas import tpu_sc as plsc

SLOPE = 0.2
CHUNK = 80
NS = 16

_MESH = plsc.VectorSubcoreMesh(core_axis_name="c", subcore_axis_name="s")
_PARAMS = pltpu.CompilerParams(needs_layout_passes=False)


def _blockdiag(a):
    H, C = a.shape
    eye = jnp.eye(H, dtype=a.dtype)
    return (a[:, :, None] * eye[:, None, :]).reshape(H * C, H)


def _dense_tc(xs, Ws, ABs):
    G, N, D = xs.shape
    RB = 2000

    def body(x_ref, w_ref, ab_ref, h_ref, abo_ref):
        x = x_ref[0]
        h = jnp.dot(x, w_ref[0], preferred_element_type=jnp.float32)
        h_ref[0] = h
        abo_ref[0] = jnp.dot(h, ab_ref[0], preferred_element_type=jnp.float32)

    return pl.pallas_call(
        body,
        grid=(G, N // RB),
        in_specs=[
            pl.BlockSpec((1, RB, D), lambda g, i: (g, i, 0)),
            pl.BlockSpec((1, D, D), lambda g, i: (g, 0, 0)),
            pl.BlockSpec((1, D, 8), lambda g, i: (g, 0, 0)),
        ],
        out_specs=[
            pl.BlockSpec((1, RB, D), lambda g, i: (g, i, 0)),
            pl.BlockSpec((1, RB, 8), lambda g, i: (g, i, 0)),
        ],
        out_shape=[
            jax.ShapeDtypeStruct((G, N, D), jnp.float32),
            jax.ShapeDtypeStruct((G, N, 8), jnp.float32),
        ],
    )(xs, Ws, ABs)


def _edge_w_sc(ab_q, src_q, dst_q, ab_u, src_u, dst_u):
    E = src_q.shape[0]
    H = 4
    NA = ab_q.shape[0]
    ept = E // NS
    WB = 2000
    n_batches = ept // WB

    @functools.partial(
        pl.kernel,
        out_type=(
            jax.ShapeDtypeStruct((E * H,), jnp.float32),
            jax.ShapeDtypeStruct((E * H,), jnp.float32),
        ),
        mesh=_MESH,
        compiler_params=_PARAMS,
        scratch_types=[
            pltpu.VMEM((NA,), jnp.float32),
            pltpu.VMEM((WB,), jnp.int32),
            pltpu.VMEM((WB,), jnp.int32),
            pltpu.VMEM((WB * H,), jnp.float32),
        ],
    )
    def k(abq_hbm, sq_hbm, dq_hbm, abu_hbm, su_hbm, du_hbm,
          wq_hbm, wu_hbm, abv, srcv, dstv, wout):
        cid = lax.axis_index("c")
        sid = lax.axis_index("s")
        lanes = lax.iota(jnp.int32, 16)

        def run(ab_hbm, src_hbm, dst_hbm, w_hbm):
            pltpu.sync_copy(ab_hbm, abv)
            ebase = sid * ept

            def ebatch(bi, _):
                off = ebase + bi * WB
                pltpu.sync_copy(src_hbm.at[pl.ds(off, WB)], srcv)
                pltpu.sync_copy(dst_hbm.at[pl.ds(off, WB)], dstv)

                def group(g, _):
                    e16 = lanes + g * 16
                    s16 = srcv[pl.ds(g * 16, 16)]
                    d16 = dstv[pl.ds(g * 16, 16)]
                    for hh in range(H):
                        a_s = plsc.load_gather(abv, [s16 * 8 + hh])
                        a_d = plsc.load_gather(abv, [d16 * 8 + (H + hh)])
                        ee = a_s + a_d
                        ee = jnp.where(ee > 0, ee, SLOPE * ee)
                        w = jnp.exp(ee)
                        plsc.store_scatter(wout, [e16 * H + hh], w)
                    return 0
                lax.fori_loop(0, WB // 16, group, 0)
                pltpu.sync_copy(wout, w_hbm.at[pl.ds(off * H, WB * H)])
                return 0
            lax.fori_loop(0, n_batches, ebatch, 0)

        @pl.when(cid == 0)
        def _():
            run(abq_hbm, sq_hbm, dq_hbm, wq_hbm)

        @pl.when(cid == 1)
        def _():
            run(abu_hbm, su_hbm, du_hbm, wu_hbm)

    return k(ab_q, src_q, dst_q, ab_u, src_u, dst_u)


def _aggregate_sc(h_q, w_q, b_q, src_q, dst_q, h_u, w_u, b_u, src_u, dst_u):
    N, D = h_q.shape
    E = src_q.shape[0]
    H = 4
    C = D // H
    ept = E // NS
    EB = 2000
    n_batches = ept // EB
    n_sub = EB // CHUNK
    row_chunks = N // CHUNK
    iters = -(-row_chunks // NS)
    DR = (N * H + 127) // 128
    DRP = -(-DR // CHUNK) * CHUNK

    @functools.partial(
        pl.kernel,
        out_type=(
            jax.ShapeDtypeStruct((N, D), jnp.float32),
            jax.ShapeDtypeStruct((N, D), jnp.float32),
        ),
        mesh=_MESH,
        compiler_params=_PARAMS,
        scratch_types=[
            pltpu.VMEM_SHARED((N, D), jnp.float32),
            pltpu.VMEM_SHARED((DRP, 128), jnp.float32),
            pltpu.VMEM((EB,), jnp.int32),
            pltpu.VMEM((EB,), jnp.int32),
            pltpu.VMEM((2 * CHUNK * H,), jnp.float32),
            pltpu.VMEM((CHUNK,), jnp.int32),
            pltpu.VMEM((CHUNK,), jnp.int32),
            pltpu.VMEM((2 * CHUNK, D), jnp.float32),
            pltpu.VMEM((CHUNK, D), jnp.float32),
            pltpu.VMEM((CHUNK, 128), jnp.float32),
            pltpu.VMEM((16, 128), jnp.float32),
            pltpu.VMEM((D,), jnp.float32),
            pltpu.SemaphoreType.DMA,
        ],
    )
    def k(hq_hbm, wq_hbm, bq_hbm, sq_hbm, dq_hbm,
          hu_hbm, wu_hbm, bu_hbm, su_hbm, du_hbm,
          outq_hbm, outu_hbm,
          acc, accd, srcb, dstb, wb, dstv, drow, hbuf, msg, mden,
          dbuf, biasv, sem):
        cid = lax.axis_index("c")
        sid = lax.axis_index("s")
        zero16 = jnp.zeros((16,), jnp.float32)
        lanes = lax.iota(jnp.int32, 16)

        def run(h_hbm, w_hbm, b_hbm, src_hbm, dst_hbm, out_hbm):
            pltpu.sync_copy(b_hbm, biasv)

            def zmsg(e, _):
                for kk in range(D // 16):
                    msg[e, pl.ds(kk * 16, 16)] = zero16
                for kk in range(128 // 16):
                    mden[e, pl.ds(kk * 16, 16)] = zero16
                return 0
            lax.fori_loop(0, CHUNK, zmsg, 0)

            def zacc(i, _):
                c = sid + i * NS

                @pl.when(c < row_chunks)
                def _():
                    pltpu.sync_copy(msg, acc.at[pl.ds(c * CHUNK, CHUNK)])
                return 0
            lax.fori_loop(0, iters, zacc, 0)

            @pl.when(sid < DRP // CHUNK)
            def _():
                pltpu.sync_copy(msg, accd.at[pl.ds(sid * CHUNK, CHUNK)])
            plsc.subcore_barrier()

            ebase = sid * ept

            def drain():
                pltpu.make_async_copy(
                    h_hbm.at[srcb.at[pl.ds(0, CHUNK)]],
                    hbuf.at[pl.ds(0, CHUNK)], sem).wait()
                pltpu.make_async_copy(
                    w_hbm.at[pl.ds(0, CHUNK * H)],
                    wb.at[pl.ds(0, CHUNK * H)], sem).wait()

            def ebatch(bi, _):
                boff = ebase + bi * EB
                pltpu.sync_copy(src_hbm.at[pl.ds(boff, EB)], srcb)
                pltpu.sync_copy(dst_hbm.at[pl.ds(boff, EB)], dstb)
                wboff = boff * H

                def fire(si, pb):
                    s0 = si * CHUNK
                    pltpu.async_copy(
                        h_hbm.at[srcb.at[pl.ds(s0, CHUNK)]],
                        hbuf.at[pl.ds(pb * CHUNK, CHUNK)], sem)
                    pltpu.async_copy(
                        w_hbm.at[pl.ds(wboff + s0 * H, CHUNK * H)],
                        wb.at[pl.ds(pb * CHUNK * H, CHUNK * H)], sem)

                fire(0, 0)

                def sub(si, _):
                    s0 = si * CHUNK
                    p = si & 1
                    ph = p * CHUNK
                    pw = p * CHUNK * H
                    drain()

                    @pl.when(si + 1 < n_sub)
                    def _():
                        fire(si + 1, 1 - p)

                    for g in range(CHUNK // 16):
                        e16 = lanes + g * 16
                        d16 = dstb[pl.ds(s0 + g * 16, 16)]
                        dstv[pl.ds(g * 16, 16)] = d16
                        drow[pl.ds(g * 16, 16)] = d16 >> 5
                        colb = (d16 & 31) * H
                        for hh in range(H):
                            w = plsc.load_gather(wb, [pw + e16 * H + hh])
                            plsc.store_scatter(mden, [e16, colb + hh], w)

                    def medge(e, _):
                        for hh in range(H):
                            wv = plsc.load_gather(
                                wb, [jnp.full((16,), e * H + hh, jnp.int32)
                                     + pw])
                            for half in range(C // 16):
                                col = hh * C + half * 16
                                msg[e, pl.ds(col, 16)] = \
                                    hbuf[ph + e, pl.ds(col, 16)] * wv
                        return 0
                    lax.fori_loop(0, CHUNK, medge, 0)
                    pltpu.sync_copy(msg, acc.at[dstv], add=True)
                    pltpu.sync_copy(mden, accd.at[drow], add=True)
                    for g in range(CHUNK // 16):
                        e16 = lanes + g * 16
                        d16 = dstb[pl.ds(s0 + g * 16, 16)]
                        colb = (d16 & 31) * H
                        for hh in range(H):
                            plsc.store_scatter(mden, [e16, colb + hh], zero16)
                    return 0
                lax.fori_loop(0, n_sub, sub, 0)
                return 0
            lax.fori_loop(0, n_batches, ebatch, 0)
            plsc.subcore_barrier()

            def nchunk(i, _):
                c = sid + i * NS

                @pl.when(c < row_chunks)
                def _():
                    base = c * CHUNK
                    pltpu.sync_copy(acc.at[pl.ds(base, CHUNK)], msg)
                    r0a = ((base // 32) // 8) * 8
                    pltpu.sync_copy(accd.at[pl.ds(r0a, 16)], dbuf)

                    def node(n, _):
                        p = (base + n) * H - r0a * 128
                        for hh in range(H):
                            rv = jnp.full((16,), (p + hh) >> 7, jnp.int32)
                            cv = jnp.full((16,), (p + hh) & 127, jnp.int32)
                            dv = plsc.load_gather(dbuf, [rv, cv])
                            invv = 1.0 / (dv + 1e-16)
                            for half in range(C // 16):
                                col = hh * C + half * 16
                                v = msg[n, pl.ds(col, 16)] * invv \
                                    + biasv[pl.ds(col, 16)]
                                hbuf[n, pl.ds(col, 16)] = jnp.maximum(v, 0.0)
                        return 0
                    lax.fori_loop(0, CHUNK, node, 0)
                    pltpu.sync_copy(hbuf.at[pl.ds(0, CHUNK)],
                                    out_hbm.at[pl.ds(base, CHUNK)])
                return 0
            lax.fori_loop(0, iters, nchunk, 0)

        @pl.when(cid == 0)
        def _():
            run(hq_hbm, wq_hbm, bq_hbm, sq_hbm, dq_hbm, outq_hbm)

        @pl.when(cid == 1)
        def _():
            run(hu_hbm, wu_hbm, bu_hbm, su_hbm, du_hbm, outu_hbm)

    return k(h_q, w_q, b_q, src_q, dst_q, h_u, w_u, b_u, src_u, dst_u)


def _session_gather_sc(pq, pu, vid_table, click_table, pos_table,
                       qidx, uidx, vidx, cidx, pidx):
    N, D = pq.shape
    BS = qidx.shape[0]
    per_tile = BS // (2 * NS)
    n_chunks = per_tile // CHUNK

    @functools.partial(
        pl.kernel,
        out_type=(
            jax.ShapeDtypeStruct((BS, D), jnp.float32),
            jax.ShapeDtypeStruct((BS, D), jnp.float32),
            jax.ShapeDtypeStruct((BS, 16), jnp.float32),
            jax.ShapeDtypeStruct((BS, 16), jnp.float32),
            jax.ShapeDtypeStruct((BS, 16), jnp.float32),
        ),
        mesh=_MESH,
        compiler_params=_PARAMS,
        scratch_types=[
            pltpu.VMEM((CHUNK,), jnp.int32),
            pltpu.VMEM((CHUNK, D), jnp.float32),
            pltpu.VMEM((CHUNK, 16), jnp.float32),
            pltpu.VMEM((16, 16), jnp.float32),
            pltpu.SemaphoreType.DMA,
        ],
    )
    def k(pq_hbm, pu_hbm, vid_hbm, click_hbm, pos_hbm,
          qi_hbm, ui_hbm, vi_hbm, ci_hbm, pi_hbm,
          oq, ou, ov, oc, op_,
          idxv, bigbuf, smallbuf, tabv, sem):
        wid = lax.axis_index("s") * 2 + lax.axis_index("c")
        base = wid * per_tile
        lanes = lax.iota(jnp.int32, 16)

        def gather_big(tab, idx_hbm, out_hbm):
            def step(j, _):
                off = base + j * CHUNK
                pltpu.sync_copy(idx_hbm.at[pl.ds(off, CHUNK)], idxv)
                pltpu.async_copy(tab.at[idxv], bigbuf, sem).wait()
                pltpu.sync_copy(bigbuf, out_hbm.at[pl.ds(off, CHUNK)])
                return 0
            lax.fori_loop(0, n_chunks, step, 0)

        def gather_small(tab, rows, idx_hbm, out_hbm):
            pltpu.sync_copy(tab, tabv.at[pl.ds(0, rows)])

            def step(j, _):
                off = base + j * CHUNK
                pltpu.sync_copy(idx_hbm.at[pl.ds(off, CHUNK)], idxv)

                def row(r, _):
                    ispl = plsc.load_gather(
                        idxv, [jnp.full((16,), r, jnp.int32)])
                    smallbuf[r, pl.ds(0, 16)] = plsc.load_gather(
                        tabv, [ispl, lanes])
                    return 0
                lax.fori_loop(0, CHUNK, row, 0)
                pltpu.sync_copy(smallbuf, out_hbm.at[pl.ds(off, CHUNK)])
                return 0
            lax.fori_loop(0, n_chunks, step, 0)

        gather_big(pq_hbm, qi_hbm, oq)
        gather_big(pu_hbm, ui_hbm, ou)
        gather_small(vid_hbm, vid_hbm.shape[0], vi_hbm, ov)
        gather_small(click_hbm, click_hbm.shape[0], ci_hbm, oc)
        gather_small(pos_hbm, pos_hbm.shape[0], pi_hbm, op_)

    return k(pq, pu, vid_table, click_table, pos_table,
             qidx, uidx, vidx, cidx, pidx)


def kernel(qid_table, uid_table, click_table, vid_table, pos_table,
           W_q, a_src_q, a_dst_q, b_q, W_u, a_src_u, a_dst_u, b_u,
           qid_edge_index, uid_edge_index, QIDS, UIDS, VIDS, CLICKS):
    B, S = QIDS.shape
    D = qid_table.shape[1]
    xs = jnp.stack([qid_table, uid_table])
    Ws = jnp.stack([W_q, W_u])
    ABs = jnp.stack([
        jnp.concatenate([_blockdiag(a_src_q), _blockdiag(a_dst_q)], axis=1),
        jnp.concatenate([_blockdiag(a_src_u), _blockdiag(a_dst_u)], axis=1),
    ])
    hs, abs_ = _dense_tc(xs, Ws, ABs)

    w_q, w_u = _edge_w_sc(
        abs_[0].reshape(-1), qid_edge_index[0], qid_edge_index[1],
        abs_[1].reshape(-1), uid_edge_index[0], uid_edge_index[1])

    pq, pu = _aggregate_sc(
        hs[0], w_q, b_q, qid_edge_index[0], qid_edge_index[1],
        hs[1], w_u, b_u, uid_edge_index[0], uid_edge_index[1])

    pidx = jnp.tile(jnp.arange(10, dtype=jnp.int32), (B * S) // 10)
    qe, ue, ve, ce, pe = _session_gather_sc(
        pq, pu, vid_table, click_table, pos_table,
        QIDS.reshape(-1), UIDS.reshape(-1), VIDS.reshape(-1),
        CLICKS.reshape(-1), pidx)
    return (qe.reshape(B, S, D), ue.reshape(B, S, D),
            ve.reshape(B, S, 16), ce.reshape(B, S, 16),
            pe.reshape(B, S, 16))


# speedup vs baseline: 46.5820x; 1.0090x over previous
import functools

import jax
import jax.numpy as jnp
from jax import lax
from jax.experimental import pallas as pl
from jax.experimental.pallas import tpu as pltpu
from jax.experimental.pallas import tpu_sc as plsc

SLOPE = 0.2
CHUNK = 80
NS = 16

_MESH = plsc.VectorSubcoreMesh(core_axis_name="c", subcore_axis_name="s")
_PARAMS = pltpu.CompilerParams(needs_layout_passes=False)


def _blockdiag(a):
    H, C = a.shape
    eye = jnp.eye(H, dtype=a.dtype)
    return (a[:, :, None] * eye[:, None, :]).reshape(H * C, H)


def _dense_tc(xs, Ws, ABs):
    G, N, D = xs.shape
    RB = 2000

    def body(x_ref, w_ref, ab_ref, h_ref, abo_ref):
        x = x_ref[0]
        h = jnp.dot(x, w_ref[0], preferred_element_type=jnp.float32)
        h_ref[0] = h
        abo_ref[0] = jnp.dot(h, ab_ref[0], preferred_element_type=jnp.float32)

    return pl.pallas_call(
        body,
        grid=(G, N // RB),
        in_specs=[
            pl.BlockSpec((1, RB, D), lambda g, i: (g, i, 0)),
            pl.BlockSpec((1, D, D), lambda g, i: (g, 0, 0)),
            pl.BlockSpec((1, D, 8), lambda g, i: (g, 0, 0)),
        ],
        out_specs=[
            pl.BlockSpec((1, RB, D), lambda g, i: (g, i, 0)),
            pl.BlockSpec((1, RB, 8), lambda g, i: (g, i, 0)),
        ],
        out_shape=[
            jax.ShapeDtypeStruct((G, N, D), jnp.float32),
            jax.ShapeDtypeStruct((G, N, 8), jnp.float32),
        ],
    )(xs, Ws, ABs)


def _edge_w_sc(ab_q, src_q, dst_q, ab_u, src_u, dst_u):
    E = src_q.shape[0]
    H = 4
    NA = ab_q.shape[0]
    ept = E // NS
    WB = 2000
    n_batches = ept // WB

    @functools.partial(
        pl.kernel,
        out_type=(
            jax.ShapeDtypeStruct((E * H,), jnp.float32),
            jax.ShapeDtypeStruct((E * H,), jnp.float32),
        ),
        mesh=_MESH,
        compiler_params=_PARAMS,
        scratch_types=[
            pltpu.VMEM((NA,), jnp.float32),
            pltpu.VMEM((WB,), jnp.int32),
            pltpu.VMEM((WB,), jnp.int32),
            pltpu.VMEM((WB * H,), jnp.float32),
        ],
    )
    def k(abq_hbm, sq_hbm, dq_hbm, abu_hbm, su_hbm, du_hbm,
          wq_hbm, wu_hbm, abv, srcv, dstv, wout):
        cid = lax.axis_index("c")
        sid = lax.axis_index("s")
        lanes = lax.iota(jnp.int32, 16)

        def run(ab_hbm, src_hbm, dst_hbm, w_hbm):
            pltpu.sync_copy(ab_hbm, abv)
            ebase = sid * ept

            def ebatch(bi, _):
                off = ebase + bi * WB
                pltpu.sync_copy(src_hbm.at[pl.ds(off, WB)], srcv)
                pltpu.sync_copy(dst_hbm.at[pl.ds(off, WB)], dstv)

                def group(g, _):
                    e16 = lanes + g * 16
                    s16 = srcv[pl.ds(g * 16, 16)]
                    d16 = dstv[pl.ds(g * 16, 16)]
                    for hh in range(H):
                        a_s = plsc.load_gather(abv, [s16 * 8 + hh])
                        a_d = plsc.load_gather(abv, [d16 * 8 + (H + hh)])
                        ee = a_s + a_d
                        ee = jnp.where(ee > 0, ee, SLOPE * ee)
                        w = jnp.exp(ee)
                        plsc.store_scatter(wout, [e16 * H + hh], w)
                    return 0
                lax.fori_loop(0, WB // 16, group, 0)
                pltpu.sync_copy(wout, w_hbm.at[pl.ds(off * H, WB * H)])
                return 0
            lax.fori_loop(0, n_batches, ebatch, 0)

        @pl.when(cid == 0)
        def _():
            run(abq_hbm, sq_hbm, dq_hbm, wq_hbm)

        @pl.when(cid == 1)
        def _():
            run(abu_hbm, su_hbm, du_hbm, wu_hbm)

    return k(ab_q, src_q, dst_q, ab_u, src_u, dst_u)


def _aggregate_sc(h_q, w_q, b_q, src_q, dst_q, h_u, w_u, b_u, src_u, dst_u):
    N, D = h_q.shape
    E = src_q.shape[0]
    H = 4
    C = D // H
    ept = E // NS
    EB = 2000
    n_batches = ept // EB
    n_sub = EB // CHUNK
    row_chunks = N // CHUNK
    iters = -(-row_chunks // NS)
    DR = (N * H + 127) // 128
    DRP = -(-DR // CHUNK) * CHUNK

    @functools.partial(
        pl.kernel,
        out_type=(
            jax.ShapeDtypeStruct((N, D), jnp.float32),
            jax.ShapeDtypeStruct((N, D), jnp.float32),
        ),
        mesh=_MESH,
        compiler_params=_PARAMS,
        scratch_types=[
            pltpu.VMEM_SHARED((N, D), jnp.float32),
            pltpu.VMEM_SHARED((DRP, 128), jnp.float32),
            pltpu.VMEM((EB,), jnp.int32),
            pltpu.VMEM((EB,), jnp.int32),
            pltpu.VMEM((2 * CHUNK * H,), jnp.float32),
            pltpu.VMEM((2, CHUNK), jnp.int32),
            pltpu.VMEM((CHUNK,), jnp.int32),
            pltpu.VMEM((2 * CHUNK, D), jnp.float32),
            pltpu.VMEM((CHUNK, D), jnp.float32),
            pltpu.VMEM((CHUNK, 128), jnp.float32),
            pltpu.VMEM((16, 128), jnp.float32),
            pltpu.VMEM((D,), jnp.float32),
            pltpu.SemaphoreType.DMA,
            pltpu.SemaphoreType.DMA,
        ],
    )
    def k(hq_hbm, wq_hbm, bq_hbm, sq_hbm, dq_hbm,
          hu_hbm, wu_hbm, bu_hbm, su_hbm, du_hbm,
          outq_hbm, outu_hbm,
          acc, accd, srcb, dstb, wb, dstv, drow, hbuf, msg, mden,
          dbuf, biasv, sem, sem2):
        cid = lax.axis_index("c")
        sid = lax.axis_index("s")
        zero16 = jnp.zeros((16,), jnp.float32)
        lanes = lax.iota(jnp.int32, 16)

        def run(h_hbm, w_hbm, b_hbm, src_hbm, dst_hbm, out_hbm):
            pltpu.sync_copy(b_hbm, biasv)

            def zmsg(e, _):
                for kk in range(D // 16):
                    msg[e, pl.ds(kk * 16, 16)] = zero16
                for kk in range(128 // 16):
                    mden[e, pl.ds(kk * 16, 16)] = zero16
                return 0
            lax.fori_loop(0, CHUNK, zmsg, 0)

            def zacc(i, _):
                c = sid + i * NS

                @pl.when(c < row_chunks)
                def _():
                    pltpu.sync_copy(msg, acc.at[pl.ds(c * CHUNK, CHUNK)])
                return 0
            lax.fori_loop(0, iters, zacc, 0)

            @pl.when(sid < DRP // CHUNK)
            def _():
                pltpu.sync_copy(msg, accd.at[pl.ds(sid * CHUNK, CHUNK)])
            plsc.subcore_barrier()

            ebase = sid * ept

            def drain():
                pltpu.make_async_copy(
                    h_hbm.at[srcb.at[pl.ds(0, CHUNK)]],
                    hbuf.at[pl.ds(0, CHUNK)], sem).wait()
                pltpu.make_async_copy(
                    w_hbm.at[pl.ds(0, CHUNK * H)],
                    wb.at[pl.ds(0, CHUNK * H)], sem).wait()

            def ebatch(bi, _):
                boff = ebase + bi * EB
                pltpu.sync_copy(src_hbm.at[pl.ds(boff, EB)], srcb)
                pltpu.sync_copy(dst_hbm.at[pl.ds(boff, EB)], dstb)
                wboff = boff * H

                def fire(si, pb):
                    s0 = si * CHUNK
                    pltpu.async_copy(
                        h_hbm.at[srcb.at[pl.ds(s0, CHUNK)]],
                        hbuf.at[pl.ds(pb * CHUNK, CHUNK)], sem)
                    pltpu.async_copy(
                        w_hbm.at[pl.ds(wboff + s0 * H, CHUNK * H)],
                        wb.at[pl.ds(pb * CHUNK * H, CHUNK * H)], sem)

                fire(0, 0)

                def sub(si, _):
                    s0 = si * CHUNK
                    p = si & 1
                    ph = p * CHUNK
                    pw = p * CHUNK * H
                    drain()

                    @pl.when(si + 1 < n_sub)
                    def _():
                        fire(si + 1, 1 - p)

                    for g in range(CHUNK // 16):
                        e16 = lanes + g * 16
                        d16 = dstb[pl.ds(s0 + g * 16, 16)]
                        dstv[p, pl.ds(g * 16, 16)] = d16
                        drow[pl.ds(g * 16, 16)] = d16 >> 5
                        colb = (d16 & 31) * H
                        for hh in range(H):
                            w = plsc.load_gather(wb, [pw + e16 * H + hh])
                            plsc.store_scatter(mden, [e16, colb + hh], w)

                    @pl.when(bi * n_sub + si > 0)
                    def _():
                        pltpu.make_async_copy(
                            msg, acc.at[dstv.at[0]], sem2).wait()

                    def medge(e, _):
                        for hh in range(H):
                            wv = plsc.load_gather(
                                wb, [jnp.full((16,), e * H + hh, jnp.int32)
                                     + pw])
                            for half in range(C // 16):
                                col = hh * C + half * 16
                                msg[e, pl.ds(col, 16)] = \
                                    hbuf[ph + e, pl.ds(col, 16)] * wv
                        return 0
                    lax.fori_loop(0, CHUNK, medge, 0)
                    pltpu.async_copy(msg, acc.at[dstv.at[p]], sem2, add=True)
                    pltpu.sync_copy(mden, accd.at[drow], add=True)
                    for g in range(CHUNK // 16):
                        e16 = lanes + g * 16
                        d16 = dstb[pl.ds(s0 + g * 16, 16)]
                        colb = (d16 & 31) * H
                        for hh in range(H):
                            plsc.store_scatter(mden, [e16, colb + hh], zero16)
                    return 0
                lax.fori_loop(0, n_sub, sub, 0)
                return 0
            lax.fori_loop(0, n_batches, ebatch, 0)
            pltpu.make_async_copy(msg, acc.at[dstv.at[0]], sem2).wait()
            plsc.subcore_barrier()

            def nchunk(i, _):
                c = sid + i * NS

                @pl.when(c < row_chunks)
                def _():
                    base = c * CHUNK
                    pltpu.sync_copy(acc.at[pl.ds(base, CHUNK)], msg)
                    r0a = ((base // 32) // 8) * 8
                    pltpu.sync_copy(accd.at[pl.ds(r0a, 16)], dbuf)

                    def node(n, _):
                        p = (base + n) * H - r0a * 128
                        for hh in range(H):
                            rv = jnp.full((16,), (p + hh) >> 7, jnp.int32)
                            cv = jnp.full((16,), (p + hh) & 127, jnp.int32)
                            dv = plsc.load_gather(dbuf, [rv, cv])
                            invv = 1.0 / (dv + 1e-16)
                            for half in range(C // 16):
                                col = hh * C + half * 16
                                v = msg[n, pl.ds(col, 16)] * invv \
                                    + biasv[pl.ds(col, 16)]
                                hbuf[n, pl.ds(col, 16)] = jnp.maximum(v, 0.0)
                        return 0
                    lax.fori_loop(0, CHUNK, node, 0)
                    pltpu.sync_copy(hbuf.at[pl.ds(0, CHUNK)],
                                    out_hbm.at[pl.ds(base, CHUNK)])
                return 0
            lax.fori_loop(0, iters, nchunk, 0)

        @pl.when(cid == 0)
        def _():
            run(hq_hbm, wq_hbm, bq_hbm, sq_hbm, dq_hbm, outq_hbm)

        @pl.when(cid == 1)
        def _():
            run(hu_hbm, wu_hbm, bu_hbm, su_hbm, du_hbm, outu_hbm)

    return k(h_q, w_q, b_q, src_q, dst_q, h_u, w_u, b_u, src_u, dst_u)


def _session_gather_sc(pq, pu, vid_table, click_table, pos_table,
                       qidx, uidx, vidx, cidx, pidx):
    N, D = pq.shape
    BS = qidx.shape[0]
    per_tile = BS // (2 * NS)
    n_chunks = per_tile // CHUNK

    @functools.partial(
        pl.kernel,
        out_type=(
            jax.ShapeDtypeStruct((BS, D), jnp.float32),
            jax.ShapeDtypeStruct((BS, D), jnp.float32),
            jax.ShapeDtypeStruct((BS, 16), jnp.float32),
            jax.ShapeDtypeStruct((BS, 16), jnp.float32),
            jax.ShapeDtypeStruct((BS, 16), jnp.float32),
        ),
        mesh=_MESH,
        compiler_params=_PARAMS,
        scratch_types=[
            pltpu.VMEM((CHUNK,), jnp.int32),
            pltpu.VMEM((CHUNK, D), jnp.float32),
            pltpu.VMEM((CHUNK, 16), jnp.float32),
            pltpu.VMEM((16, 16), jnp.float32),
            pltpu.SemaphoreType.DMA,
        ],
    )
    def k(pq_hbm, pu_hbm, vid_hbm, click_hbm, pos_hbm,
          qi_hbm, ui_hbm, vi_hbm, ci_hbm, pi_hbm,
          oq, ou, ov, oc, op_,
          idxv, bigbuf, smallbuf, tabv, sem):
        wid = lax.axis_index("s") * 2 + lax.axis_index("c")
        base = wid * per_tile
        lanes = lax.iota(jnp.int32, 16)

        def gather_big(tab, idx_hbm, out_hbm):
            def step(j, _):
                off = base + j * CHUNK
                pltpu.sync_copy(idx_hbm.at[pl.ds(off, CHUNK)], idxv)
                pltpu.async_copy(tab.at[idxv], bigbuf, sem).wait()
                pltpu.sync_copy(bigbuf, out_hbm.at[pl.ds(off, CHUNK)])
                return 0
            lax.fori_loop(0, n_chunks, step, 0)

        def gather_small(tab, rows, idx_hbm, out_hbm):
            pltpu.sync_copy(tab, tabv.at[pl.ds(0, rows)])

            def step(j, _):
                off = base + j * CHUNK
                pltpu.sync_copy(idx_hbm.at[pl.ds(off, CHUNK)], idxv)

                def row(r, _):
                    ispl = plsc.load_gather(
                        idxv, [jnp.full((16,), r, jnp.int32)])
                    smallbuf[r, pl.ds(0, 16)] = plsc.load_gather(
                        tabv, [ispl, lanes])
                    return 0
                lax.fori_loop(0, CHUNK, row, 0)
                pltpu.sync_copy(smallbuf, out_hbm.at[pl.ds(off, CHUNK)])
                return 0
            lax.fori_loop(0, n_chunks, step, 0)

        gather_big(pq_hbm, qi_hbm, oq)
        gather_big(pu_hbm, ui_hbm, ou)
        gather_small(vid_hbm, vid_hbm.shape[0], vi_hbm, ov)
        gather_small(click_hbm, click_hbm.shape[0], ci_hbm, oc)
        gather_small(pos_hbm, pos_hbm.shape[0], pi_hbm, op_)

    return k(pq, pu, vid_table, click_table, pos_table,
             qidx, uidx, vidx, cidx, pidx)


def kernel(qid_table, uid_table, click_table, vid_table, pos_table,
           W_q, a_src_q, a_dst_q, b_q, W_u, a_src_u, a_dst_u, b_u,
           qid_edge_index, uid_edge_index, QIDS, UIDS, VIDS, CLICKS):
    B, S = QIDS.shape
    D = qid_table.shape[1]
    xs = jnp.stack([qid_table, uid_table])
    Ws = jnp.stack([W_q, W_u])
    ABs = jnp.stack([
        jnp.concatenate([_blockdiag(a_src_q), _blockdiag(a_dst_q)], axis=1),
        jnp.concatenate([_blockdiag(a_src_u), _blockdiag(a_dst_u)], axis=1),
    ])
    hs, abs_ = _dense_tc(xs, Ws, ABs)

    w_q, w_u = _edge_w_sc(
        abs_[0].reshape(-1), qid_edge_index[0], qid_edge_index[1],
        abs_[1].reshape(-1), uid_edge_index[0], uid_edge_index[1])

    pq, pu = _aggregate_sc(
        hs[0], w_q, b_q, qid_edge_index[0], qid_edge_index[1],
        hs[1], w_u, b_u, uid_edge_index[0], uid_edge_index[1])

    pidx = jnp.tile(jnp.arange(10, dtype=jnp.int32), (B * S) // 10)
    qe, ue, ve, ce, pe = _session_gather_sc(
        pq, pu, vid_table, click_table, pos_table,
        QIDS.reshape(-1), UIDS.reshape(-1), VIDS.reshape(-1),
        CLICKS.reshape(-1), pidx)
    return (qe.reshape(B, S, D), ue.reshape(B, S, D),
            ve.reshape(B, S, 16), ce.reshape(B, S, 16),
            pe.reshape(B, S, 16))


# speedup vs baseline: 50.4499x; 1.0830x over previous
import functools

import jax
import jax.numpy as jnp
from jax import lax
from jax.experimental import pallas as pl
from jax.experimental.pallas import tpu as pltpu
from jax.experimental.pallas import tpu_sc as plsc

SLOPE = 0.2
CHUNK = 80
NS = 16

_MESH = plsc.VectorSubcoreMesh(core_axis_name="c", subcore_axis_name="s")
_PARAMS = pltpu.CompilerParams(needs_layout_passes=False)


def _blockdiag(a):
    H, C = a.shape
    eye = jnp.eye(H, dtype=a.dtype)
    return (a[:, :, None] * eye[:, None, :]).reshape(H * C, H)


def _dense_tc(xs, Ws, ABs):
    G, N, D = xs.shape
    RB = 2000

    def body(x_ref, w_ref, ab_ref, h_ref, abo_ref):
        x = x_ref[0]
        h = jnp.dot(x, w_ref[0], preferred_element_type=jnp.float32)
        h_ref[0] = h
        abo_ref[0] = jnp.dot(h, ab_ref[0], preferred_element_type=jnp.float32)

    return pl.pallas_call(
        body,
        grid=(G, N // RB),
        in_specs=[
            pl.BlockSpec((1, RB, D), lambda g, i: (g, i, 0)),
            pl.BlockSpec((1, D, D), lambda g, i: (g, 0, 0)),
            pl.BlockSpec((1, D, 8), lambda g, i: (g, 0, 0)),
        ],
        out_specs=[
            pl.BlockSpec((1, RB, D), lambda g, i: (g, i, 0)),
            pl.BlockSpec((1, RB, 8), lambda g, i: (g, i, 0)),
        ],
        out_shape=[
            jax.ShapeDtypeStruct((G, N, D), jnp.float32),
            jax.ShapeDtypeStruct((G, N, 8), jnp.float32),
        ],
    )(xs, Ws, ABs)


def _edge_w_sc(ab_q, src_q, dst_q, ab_u, src_u, dst_u):
    E = src_q.shape[0]
    H = 4
    NA = ab_q.shape[0]
    ept = E // NS
    WB = 2000
    n_batches = ept // WB

    @functools.partial(
        pl.kernel,
        out_type=(
            jax.ShapeDtypeStruct((E * H,), jnp.float32),
            jax.ShapeDtypeStruct((E * H,), jnp.float32),
        ),
        mesh=_MESH,
        compiler_params=_PARAMS,
        scratch_types=[
            pltpu.VMEM((NA,), jnp.float32),
            pltpu.VMEM((WB,), jnp.int32),
            pltpu.VMEM((WB,), jnp.int32),
            pltpu.VMEM((WB * H,), jnp.float32),
        ],
    )
    def k(abq_hbm, sq_hbm, dq_hbm, abu_hbm, su_hbm, du_hbm,
          wq_hbm, wu_hbm, abv, srcv, dstv, wout):
        cid = lax.axis_index("c")
        sid = lax.axis_index("s")
        lanes = lax.iota(jnp.int32, 16)

        def run(ab_hbm, src_hbm, dst_hbm, w_hbm):
            pltpu.sync_copy(ab_hbm, abv)
            ebase = sid * ept

            def ebatch(bi, _):
                off = ebase + bi * WB
                pltpu.sync_copy(src_hbm.at[pl.ds(off, WB)], srcv)
                pltpu.sync_copy(dst_hbm.at[pl.ds(off, WB)], dstv)

                def group(g, _):
                    e16 = lanes + g * 16
                    s16 = srcv[pl.ds(g * 16, 16)]
                    d16 = dstv[pl.ds(g * 16, 16)]
                    for hh in range(H):
                        a_s = plsc.load_gather(abv, [s16 * 8 + hh])
                        a_d = plsc.load_gather(abv, [d16 * 8 + (H + hh)])
                        ee = a_s + a_d
                        ee = jnp.where(ee > 0, ee, SLOPE * ee)
                        w = jnp.exp(ee)
                        plsc.store_scatter(wout, [e16 * H + hh], w)
                    return 0
                lax.fori_loop(0, WB // 16, group, 0)
                pltpu.sync_copy(wout, w_hbm.at[pl.ds(off * H, WB * H)])
                return 0
            lax.fori_loop(0, n_batches, ebatch, 0)

        @pl.when(cid == 0)
        def _():
            run(abq_hbm, sq_hbm, dq_hbm, wq_hbm)

        @pl.when(cid == 1)
        def _():
            run(abu_hbm, su_hbm, du_hbm, wu_hbm)

    return k(ab_q, src_q, dst_q, ab_u, src_u, dst_u)


def _aggregate_sc(h_q, w_q, b_q, src_q, dst_q, h_u, w_u, b_u, src_u, dst_u):
    N, D = h_q.shape
    E = src_q.shape[0]
    H = 4
    C = D // H
    ept = E // NS
    EB = 2000
    n_batches = ept // EB
    n_sub = EB // CHUNK
    row_chunks = N // CHUNK
    iters = -(-row_chunks // NS)
    DR = (N * H + 127) // 128
    DRP = -(-DR // CHUNK) * CHUNK

    @functools.partial(
        pl.kernel,
        out_type=(
            jax.ShapeDtypeStruct((N, D), jnp.float32),
            jax.ShapeDtypeStruct((N, D), jnp.float32),
        ),
        mesh=_MESH,
        compiler_params=_PARAMS,
        scratch_types=[
            pltpu.VMEM_SHARED((N, D), jnp.float32),
            pltpu.VMEM_SHARED((DRP, 128), jnp.float32),
            pltpu.VMEM((EB,), jnp.int32),
            pltpu.VMEM((EB,), jnp.int32),
            pltpu.VMEM((2 * CHUNK * H,), jnp.float32),
            pltpu.VMEM((2, CHUNK), jnp.int32),
            pltpu.VMEM((CHUNK,), jnp.int32),
            pltpu.VMEM((2 * CHUNK, D), jnp.float32),
            pltpu.VMEM((CHUNK, D), jnp.float32),
            pltpu.VMEM((CHUNK, 128), jnp.float32),
            pltpu.VMEM((16, 128), jnp.float32),
            pltpu.VMEM((D,), jnp.float32),
            pltpu.SemaphoreType.DMA,
            pltpu.SemaphoreType.DMA,
            pltpu.SemaphoreType.DMA,
        ],
    )
    def k(hq_hbm, wq_hbm, bq_hbm, sq_hbm, dq_hbm,
          hu_hbm, wu_hbm, bu_hbm, su_hbm, du_hbm,
          outq_hbm, outu_hbm,
          acc, accd, srcb, dstb, wb, dstv, drow, hbuf, msg, mden,
          dbuf, biasv, sem, sem2, sem3):
        cid = lax.axis_index("c")
        sid = lax.axis_index("s")
        zero16 = jnp.zeros((16,), jnp.float32)
        lanes = lax.iota(jnp.int32, 16)

        def run(h_hbm, w_hbm, b_hbm, src_hbm, dst_hbm, out_hbm):
            pltpu.sync_copy(b_hbm, biasv)

            def zmsg(e, _):
                for kk in range(D // 16):
                    msg[e, pl.ds(kk * 16, 16)] = zero16
                for kk in range(128 // 16):
                    mden[e, pl.ds(kk * 16, 16)] = zero16
                return 0
            lax.fori_loop(0, CHUNK, zmsg, 0)

            def zacc(i, _):
                c = sid + i * NS

                @pl.when(c < row_chunks)
                def _():
                    pltpu.sync_copy(msg, acc.at[pl.ds(c * CHUNK, CHUNK)])
                return 0
            lax.fori_loop(0, iters, zacc, 0)

            @pl.when(sid < DRP // CHUNK)
            def _():
                pltpu.sync_copy(msg, accd.at[pl.ds(sid * CHUNK, CHUNK)])
            plsc.subcore_barrier()

            ebase = sid * ept

            def drain():
                pltpu.make_async_copy(
                    h_hbm.at[srcb.at[pl.ds(0, CHUNK)]],
                    hbuf.at[pl.ds(0, CHUNK)], sem).wait()
                pltpu.make_async_copy(
                    w_hbm.at[pl.ds(0, CHUNK * H)],
                    wb.at[pl.ds(0, CHUNK * H)], sem).wait()

            def ebatch(bi, _):
                boff = ebase + bi * EB
                pltpu.sync_copy(src_hbm.at[pl.ds(boff, EB)], srcb)
                pltpu.sync_copy(dst_hbm.at[pl.ds(boff, EB)], dstb)
                wboff = boff * H

                def fire(si, pb):
                    s0 = si * CHUNK
                    pltpu.async_copy(
                        h_hbm.at[srcb.at[pl.ds(s0, CHUNK)]],
                        hbuf.at[pl.ds(pb * CHUNK, CHUNK)], sem)
                    pltpu.async_copy(
                        w_hbm.at[pl.ds(wboff + s0 * H, CHUNK * H)],
                        wb.at[pl.ds(pb * CHUNK * H, CHUNK * H)], sem)

                fire(0, 0)

                def sub(si, _):
                    s0 = si * CHUNK
                    p = si & 1
                    ph = p * CHUNK
                    pw = p * CHUNK * H
                    drain()

                    @pl.when(si + 1 < n_sub)
                    def _():
                        fire(si + 1, 1 - p)

                    for g in range(CHUNK // 16):
                        e16 = lanes + g * 16
                        d16 = dstb[pl.ds(s0 + g * 16, 16)]
                        dstv[p, pl.ds(g * 16, 16)] = d16
                        drow[pl.ds(g * 16, 16)] = d16 >> 5
                        colb = (d16 & 31) * H
                        for hh in range(H):
                            w = plsc.load_gather(wb, [pw + e16 * H + hh])
                            plsc.store_scatter(mden, [e16, colb + hh], w)

                    @pl.when(bi * n_sub + si > 0)
                    def _():
                        pltpu.make_async_copy(
                            msg, acc.at[dstv.at[0]], sem2).wait()

                    pltpu.async_copy(mden, accd.at[drow], sem3, add=True)

                    def medge(e, _):
                        for hh in range(H):
                            wv = plsc.load_gather(
                                wb, [jnp.full((16,), e * H + hh, jnp.int32)
                                     + pw])
                            for half in range(C // 16):
                                col = hh * C + half * 16
                                msg[e, pl.ds(col, 16)] = \
                                    hbuf[ph + e, pl.ds(col, 16)] * wv
                        return 0
                    lax.fori_loop(0, CHUNK, medge, 0)
                    pltpu.async_copy(msg, acc.at[dstv.at[p]], sem2, add=True)
                    pltpu.make_async_copy(mden, accd.at[drow], sem3).wait()
                    for g in range(CHUNK // 16):
                        e16 = lanes + g * 16
                        d16 = dstb[pl.ds(s0 + g * 16, 16)]
                        colb = (d16 & 31) * H
                        for hh in range(H):
                            plsc.store_scatter(mden, [e16, colb + hh], zero16)
                    return 0
                lax.fori_loop(0, n_sub, sub, 0)
                return 0
            lax.fori_loop(0, n_batches, ebatch, 0)
            pltpu.make_async_copy(msg, acc.at[dstv.at[0]], sem2).wait()
            plsc.subcore_barrier()

            def nchunk(i, _):
                c = sid + i * NS

                @pl.when(c < row_chunks)
                def _():
                    base = c * CHUNK
                    pltpu.sync_copy(acc.at[pl.ds(base, CHUNK)], msg)
                    r0a = ((base // 32) // 8) * 8
                    pltpu.sync_copy(accd.at[pl.ds(r0a, 16)], dbuf)

                    def node(n, _):
                        p = (base + n) * H - r0a * 128
                        for hh in range(H):
                            rv = jnp.full((16,), (p + hh) >> 7, jnp.int32)
                            cv = jnp.full((16,), (p + hh) & 127, jnp.int32)
                            dv = plsc.load_gather(dbuf, [rv, cv])
                            invv = 1.0 / (dv + 1e-16)
                            for half in range(C // 16):
                                col = hh * C + half * 16
                                v = msg[n, pl.ds(col, 16)] * invv \
                                    + biasv[pl.ds(col, 16)]
                                hbuf[n, pl.ds(col, 16)] = jnp.maximum(v, 0.0)
                        return 0
                    lax.fori_loop(0, CHUNK, node, 0)
                    pltpu.sync_copy(hbuf.at[pl.ds(0, CHUNK)],
                                    out_hbm.at[pl.ds(base, CHUNK)])
                return 0
            lax.fori_loop(0, iters, nchunk, 0)

        @pl.when(cid == 0)
        def _():
            run(hq_hbm, wq_hbm, bq_hbm, sq_hbm, dq_hbm, outq_hbm)

        @pl.when(cid == 1)
        def _():
            run(hu_hbm, wu_hbm, bu_hbm, su_hbm, du_hbm, outu_hbm)

    return k(h_q, w_q, b_q, src_q, dst_q, h_u, w_u, b_u, src_u, dst_u)


def _session_gather_sc(pq, pu, vid_table, click_table, pos_table,
                       qidx, uidx, vidx, cidx, pidx):
    N, D = pq.shape
    BS = qidx.shape[0]
    per_tile = BS // (2 * NS)
    n_chunks = per_tile // CHUNK

    @functools.partial(
        pl.kernel,
        out_type=(
            jax.ShapeDtypeStruct((BS, D), jnp.float32),
            jax.ShapeDtypeStruct((BS, D), jnp.float32),
            jax.ShapeDtypeStruct((BS, 16), jnp.float32),
            jax.ShapeDtypeStruct((BS, 16), jnp.float32),
            jax.ShapeDtypeStruct((BS, 16), jnp.float32),
        ),
        mesh=_MESH,
        compiler_params=_PARAMS,
        scratch_types=[
            pltpu.VMEM((CHUNK,), jnp.int32),
            pltpu.VMEM((CHUNK, D), jnp.float32),
            pltpu.VMEM((CHUNK, 16), jnp.float32),
            pltpu.VMEM((16, 16), jnp.float32),
            pltpu.SemaphoreType.DMA,
        ],
    )
    def k(pq_hbm, pu_hbm, vid_hbm, click_hbm, pos_hbm,
          qi_hbm, ui_hbm, vi_hbm, ci_hbm, pi_hbm,
          oq, ou, ov, oc, op_,
          idxv, bigbuf, smallbuf, tabv, sem):
        wid = lax.axis_index("s") * 2 + lax.axis_index("c")
        base = wid * per_tile
        lanes = lax.iota(jnp.int32, 16)

        def gather_big(tab, idx_hbm, out_hbm):
            def step(j, _):
                off = base + j * CHUNK
                pltpu.sync_copy(idx_hbm.at[pl.ds(off, CHUNK)], idxv)
                pltpu.async_copy(tab.at[idxv], bigbuf, sem).wait()
                pltpu.sync_copy(bigbuf, out_hbm.at[pl.ds(off, CHUNK)])
                return 0
            lax.fori_loop(0, n_chunks, step, 0)

        def gather_small(tab, rows, idx_hbm, out_hbm):
            pltpu.sync_copy(tab, tabv.at[pl.ds(0, rows)])

            def step(j, _):
                off = base + j * CHUNK
                pltpu.sync_copy(idx_hbm.at[pl.ds(off, CHUNK)], idxv)

                def row(r, _):
                    ispl = plsc.load_gather(
                        idxv, [jnp.full((16,), r, jnp.int32)])
                    smallbuf[r, pl.ds(0, 16)] = plsc.load_gather(
                        tabv, [ispl, lanes])
                    return 0
                lax.fori_loop(0, CHUNK, row, 0)
                pltpu.sync_copy(smallbuf, out_hbm.at[pl.ds(off, CHUNK)])
                return 0
            lax.fori_loop(0, n_chunks, step, 0)

        gather_big(pq_hbm, qi_hbm, oq)
        gather_big(pu_hbm, ui_hbm, ou)
        gather_small(vid_hbm, vid_hbm.shape[0], vi_hbm, ov)
        gather_small(click_hbm, click_hbm.shape[0], ci_hbm, oc)
        gather_small(pos_hbm, pos_hbm.shape[0], pi_hbm, op_)

    return k(pq, pu, vid_table, click_table, pos_table,
             qidx, uidx, vidx, cidx, pidx)


def kernel(qid_table, uid_table, click_table, vid_table, pos_table,
           W_q, a_src_q, a_dst_q, b_q, W_u, a_src_u, a_dst_u, b_u,
           qid_edge_index, uid_edge_index, QIDS, UIDS, VIDS, CLICKS):
    B, S = QIDS.shape
    D = qid_table.shape[1]
    xs = jnp.stack([qid_table, uid_table])
    Ws = jnp.stack([W_q, W_u])
    ABs = jnp.stack([
        jnp.concatenate([_blockdiag(a_src_q), _blockdiag(a_dst_q)], axis=1),
        jnp.concatenate([_blockdiag(a_src_u), _blockdiag(a_dst_u)], axis=1),
    ])
    hs, abs_ = _dense_tc(xs, Ws, ABs)

    w_q, w_u = _edge_w_sc(
        abs_[0].reshape(-1), qid_edge_index[0], qid_edge_index[1],
        abs_[1].reshape(-1), uid_edge_index[0], uid_edge_index[1])

    pq, pu = _aggregate_sc(
        hs[0], w_q, b_q, qid_edge_index[0], qid_edge_index[1],
        hs[1], w_u, b_u, uid_edge_index[0], uid_edge_index[1])

    pidx = jnp.tile(jnp.arange(10, dtype=jnp.int32), (B * S) // 10)
    qe, ue, ve, ce, pe = _session_gather_sc(
        pq, pu, vid_table, click_table, pos_table,
        QIDS.reshape(-1), UIDS.reshape(-1), VIDS.reshape(-1),
        CLICKS.reshape(-1), pidx)
    return (qe.reshape(B, S, D), ue.reshape(B, S, D),
            ve.reshape(B, S, 16), ce.reshape(B, S, 16),
            pe.reshape(B, S, 16))
